# Initial kernel scaffold; baseline (speedup 1.0000x reference)
import jax
import jax.numpy as jnp
from jax.experimental import pallas as pl


def kernel(features, edge_index, W_conv, b_conv, W_aggr, b_aggr):
    raise NotImplementedError("write your pallas kernel here")


# speedup vs baseline: 5.8640x; 5.8640x over previous
import functools

import jax
import jax.numpy as jnp
from jax import lax
from jax.experimental import pallas as pl
from jax.experimental.pallas import tpu as pltpu
from jax.experimental.pallas import tpu_sc as plsc

NC = 2
NS = 16
NW = NC * NS
CH = 128


def _mesh():
    return plsc.VectorSubcoreMesh(
        core_axis_name="c", subcore_axis_name="s", num_cores=NC, num_subcores=NS
    )


def _build_deg(E, N_pad):
    n_chunks = E // CH
    base_per, extra = divmod(n_chunks, NW)
    spt = N_pad // NS

    @functools.partial(
        pl.kernel,
        mesh=_mesh(),
        out_type=jax.ShapeDtypeStruct((NC * 2 * N_pad,), jnp.float32),
        scratch_types=[
            pltpu.VMEM((CH,), jnp.int32),
            pltpu.VMEM((CH,), jnp.float32),
            pltpu.VMEM((spt,), jnp.float32),
            pltpu.VMEM_SHARED((N_pad,), jnp.float32),
            pltpu.VMEM_SHARED((N_pad,), jnp.float32),
        ],
    )
    def deg_kernel(src_hbm, dst_hbm, out_hbm, idx_v, ones_v, zslice_v, dsrc_sh, ddst_sh):
        cid = lax.axis_index("c")
        sid = lax.axis_index("s")
        w = cid * NS + sid

        @pl.loop(0, CH // 16)
        def _(i):
            ones_v[pl.ds(i * 16, 16)] = jnp.full((16,), 1.0, jnp.float32)

        @pl.loop(0, spt // 16)
        def _(i):
            zslice_v[pl.ds(i * 16, 16)] = jnp.zeros((16,), jnp.float32)

        nb = sid * spt
        pltpu.sync_copy(zslice_v, dsrc_sh.at[pl.ds(nb, spt)])
        pltpu.sync_copy(zslice_v, ddst_sh.at[pl.ds(nb, spt)])
        plsc.subcore_barrier()

        nj = base_per + jnp.where(w < extra, 1, 0)

        @pl.loop(0, nj)
        def _(j):
            g = (w + NW * j) * CH
            pltpu.sync_copy(src_hbm.at[pl.ds(g, CH)], idx_v)
            pltpu.sync_copy(ones_v, dsrc_sh.at[idx_v], add=True)
            pltpu.sync_copy(dst_hbm.at[pl.ds(g, CH)], idx_v)
            pltpu.sync_copy(ones_v, ddst_sh.at[idx_v], add=True)

        plsc.subcore_barrier()
        pltpu.sync_copy(
            dsrc_sh.at[pl.ds(nb, spt)],
            out_hbm.at[pl.ds((cid * 2 + 0) * N_pad + nb, spt)],
        )
        pltpu.sync_copy(
            ddst_sh.at[pl.ds(nb, spt)],
            out_hbm.at[pl.ds((cid * 2 + 1) * N_pad + nb, spt)],
        )

    return deg_kernel


def _build_agg(E, N_pad, D):
    n_chunks = E // CH
    base_per, extra = divmod(n_chunks, NW)
    rpt = N_pad // NS
    ZR = 128

    @functools.partial(
        pl.kernel,
        mesh=_mesh(),
        out_type=jax.ShapeDtypeStruct((NC * N_pad, D), jnp.float32),
        scratch_types=[
            pltpu.VMEM((CH,), jnp.int32),
            pltpu.VMEM((CH,), jnp.int32),
            pltpu.VMEM((CH, D), jnp.float32),
            pltpu.VMEM((ZR, D), jnp.float32),
            pltpu.VMEM_SHARED((N_pad, D), jnp.float32),
            pltpu.SemaphoreType.DMA,
        ],
    )
    def agg_kernel(h_hbm, src_hbm, dst_hbm, out_hbm, sidx_v, didx_v, rows_v, zero_v, acc_sh, sem):
        cid = lax.axis_index("c")
        sid = lax.axis_index("s")
        w = cid * NS + sid

        @pl.loop(0, ZR)
        def _(r):
            @pl.loop(0, D // 16)
            def _(k):
                zero_v[r, pl.ds(k * 16, 16)] = jnp.zeros((16,), jnp.float32)

        rb = sid * rpt

        @pl.loop(0, rpt // ZR)
        def _(t):
            pltpu.sync_copy(zero_v, acc_sh.at[pl.ds(rb + t * ZR, ZR)])

        plsc.subcore_barrier()

        nj = base_per + jnp.where(w < extra, 1, 0)

        @pl.loop(0, nj)
        def _(j):
            g = (w + NW * j) * CH
            pltpu.sync_copy(src_hbm.at[pl.ds(g, CH)], sidx_v)
            pltpu.sync_copy(dst_hbm.at[pl.ds(g, CH)], didx_v)
            pltpu.async_copy(h_hbm.at[sidx_v], rows_v, sem).wait()
            pltpu.sync_copy(rows_v, acc_sh.at[didx_v], add=True)

        plsc.subcore_barrier()
        pltpu.sync_copy(
            acc_sh.at[pl.ds(rb, rpt)],
            out_hbm.at[pl.ds(cid * N_pad + rb, rpt)],
        )

    return agg_kernel


def _h_body(deg_ref, f_ref, h_ref):
    dsrc = deg_ref[0, 0, :] + deg_ref[1, 0, :]
    norm = lax.rsqrt(jnp.maximum(dsrc, 1.0))
    h_ref[...] = f_ref[...] * norm[:, None]


def _final_body(aggp_ref, deg_ref, f_ref, wc_ref, bc_ref, wa_ref, ba_ref, o_ref):
    agg = aggp_ref[0] + aggp_ref[1]
    din = deg_ref[0, 1, :] + deg_ref[1, 1, :]
    norm = lax.rsqrt(jnp.maximum(din, 1.0))
    agg = agg * norm[:, None]
    conv = jnp.dot(agg, wc_ref[...], preferred_element_type=jnp.float32)
    conv = conv + bc_ref[...][None, :]
    D = f_ref.shape[1]
    out = jnp.dot(conv, wa_ref[0:D, :], preferred_element_type=jnp.float32)
    out = out + jnp.dot(f_ref[...], wa_ref[D : 2 * D, :], preferred_element_type=jnp.float32)
    o_ref[...] = out + ba_ref[...][None, :]


def kernel(features, edge_index, W_conv, b_conv, W_aggr, b_aggr):
    N, D = features.shape
    E = edge_index.shape[1]
    N_pad = ((N + 2 * NW * 16 - 1) // (2 * NW * 16)) * (2 * NW * 16)
    BLK = 1024

    src = edge_index[0]
    dst = edge_index[1]

    deg = _build_deg(E, N_pad)(src, dst).reshape(NC, 2, N_pad)
    fpad = jnp.pad(features, ((0, N_pad - N), (0, 0)))

    h = pl.pallas_call(
        _h_body,
        grid=(N_pad // BLK,),
        in_specs=[
            pl.BlockSpec((NC, 2, BLK), lambda i: (0, 0, i)),
            pl.BlockSpec((BLK, D), lambda i: (i, 0)),
        ],
        out_specs=pl.BlockSpec((BLK, D), lambda i: (i, 0)),
        out_shape=jax.ShapeDtypeStruct((N_pad, D), jnp.float32),
    )(deg, fpad)

    aggp = _build_agg(E, N_pad, D)(h, src, dst).reshape(NC, N_pad, D)

    out = pl.pallas_call(
        _final_body,
        grid=(N_pad // BLK,),
        in_specs=[
            pl.BlockSpec((NC, BLK, D), lambda i: (0, i, 0)),
            pl.BlockSpec((NC, 2, BLK), lambda i: (0, 0, i)),
            pl.BlockSpec((BLK, D), lambda i: (i, 0)),
            pl.BlockSpec((D, D), lambda i: (0, 0)),
            pl.BlockSpec((D,), lambda i: (0,)),
            pl.BlockSpec((2 * D, D), lambda i: (0, 0)),
            pl.BlockSpec((D,), lambda i: (0,)),
        ],
        out_specs=pl.BlockSpec((BLK, D), lambda i: (i, 0)),
        out_shape=jax.ShapeDtypeStruct((N_pad, D), jnp.float32),
    )(aggp, deg, fpad, W_conv, b_conv, W_aggr, b_aggr)

    return out[:N]


# speedup vs baseline: 10.6613x; 1.8181x over previous
import functools

import jax
import jax.numpy as jnp
from jax import lax
from jax.experimental import pallas as pl
from jax.experimental.pallas import tpu as pltpu
from jax.experimental.pallas import tpu_sc as plsc

NC = 2
NS = 16
NW = NC * NS
CH = 128
KR = 80
NBUF = 2


def _mesh():
    return plsc.VectorSubcoreMesh(
        core_axis_name="c", subcore_axis_name="s", num_cores=NC, num_subcores=NS
    )


def _build_deg(N_pad):
    spt = N_pad // NS
    BK = 8

    @functools.partial(
        pl.kernel,
        mesh=_mesh(),
        out_type=jax.ShapeDtypeStruct((NC * 2 * N_pad,), jnp.float32),
        scratch_types=[
            pltpu.VMEM((KR, CH), jnp.int32),
            pltpu.VMEM((KR, CH), jnp.int32),
            pltpu.VMEM((CH,), jnp.float32),
            pltpu.VMEM((spt,), jnp.float32),
            pltpu.VMEM_SHARED((N_pad,), jnp.float32),
            pltpu.VMEM_SHARED((N_pad,), jnp.float32),
            pltpu.SemaphoreType.DMA,
            pltpu.SemaphoreType.DMA,
        ],
    )
    def deg_kernel(src_hbm, dst_hbm, out_hbm, sidx_v, didx_v, ones_v, zslice_v,
                   dsrc_sh, ddst_sh, lsem, ssem):
        cid = lax.axis_index("c")
        sid = lax.axis_index("s")
        w = cid * NS + sid

        ld_s = pltpu.async_copy(src_hbm.at[pl.ds(w * KR, KR)], sidx_v, lsem)
        ld_d = pltpu.async_copy(dst_hbm.at[pl.ds(w * KR, KR)], didx_v, lsem)

        @pl.loop(0, CH // 16)
        def _(i):
            ones_v[pl.ds(i * 16, 16)] = jnp.full((16,), 1.0, jnp.float32)

        @pl.loop(0, spt // 16)
        def _(i):
            zslice_v[pl.ds(i * 16, 16)] = jnp.zeros((16,), jnp.float32)

        nb = sid * spt
        pltpu.sync_copy(zslice_v, dsrc_sh.at[pl.ds(nb, spt)])
        pltpu.sync_copy(zslice_v, ddst_sh.at[pl.ds(nb, spt)])
        ld_s.wait()
        ld_d.wait()
        plsc.subcore_barrier()

        @pl.loop(0, KR // BK)
        def _(t):
            descs = []
            for b in range(BK):
                descs.append(pltpu.async_copy(
                    ones_v, dsrc_sh.at[sidx_v.at[t * BK + b]], ssem, add=True))
                descs.append(pltpu.async_copy(
                    ones_v, ddst_sh.at[didx_v.at[t * BK + b]], ssem, add=True))
            for d in descs:
                d.wait()

        plsc.subcore_barrier()
        pltpu.sync_copy(
            dsrc_sh.at[pl.ds(nb, spt)],
            out_hbm.at[pl.ds((cid * 2 + 0) * N_pad + nb, spt)],
        )
        pltpu.sync_copy(
            ddst_sh.at[pl.ds(nb, spt)],
            out_hbm.at[pl.ds((cid * 2 + 1) * N_pad + nb, spt)],
        )

    return deg_kernel


def _build_agg(N_pad, D):
    rpt = N_pad // NS
    IB = 16
    NB_I = KR // IB

    @functools.partial(
        pl.kernel,
        mesh=_mesh(),
        out_type=jax.ShapeDtypeStruct((NC * N_pad, D), jnp.float32),
        scratch_types=[
            pltpu.VMEM((2, IB, CH), jnp.int32),
            pltpu.VMEM((2, IB, CH), jnp.int32),
            pltpu.VMEM((NBUF, CH, D), jnp.float32),
            pltpu.VMEM_SHARED((N_pad, D), jnp.float32),
            pltpu.SemaphoreType.DMA,
            pltpu.SemaphoreType.DMA,
        ]
        + [pltpu.SemaphoreType.DMA] * NBUF
        + [pltpu.SemaphoreType.DMA] * NBUF,
    )
    def agg_kernel(h_hbm, src_hbm, dst_hbm, out_hbm, sidx_v, didx_v, rows_v,
                   acc_sh, *sems):
        lsem = sems[:2]
        gsem = sems[2 : 2 + NBUF]
        ssem = sems[2 + NBUF :]
        cid = lax.axis_index("c")
        sid = lax.axis_index("s")
        w = cid * NS + sid
        rowbase = w * KR

        pltpu.async_copy(src_hbm.at[pl.ds(rowbase, IB)], sidx_v.at[0], lsem[0])
        pltpu.async_copy(dst_hbm.at[pl.ds(rowbase, IB)], didx_v.at[0], lsem[0])

        @pl.loop(0, CH)
        def _(r):
            for k in range(D // 16):
                rows_v[0, r, pl.ds(k * 16, 16)] = jnp.zeros((16,), jnp.float32)

        rb = sid * rpt

        @pl.loop(0, rpt // CH)
        def _(t):
            pltpu.sync_copy(rows_v.at[0], acc_sh.at[pl.ds(rb + t * CH, CH)])

        plsc.subcore_barrier()

        for tb in range(NB_I):
            slot = tb % 2
            pltpu.make_async_copy(
                src_hbm.at[pl.ds(0, IB)], sidx_v.at[slot], lsem[slot]
            ).wait()
            pltpu.make_async_copy(
                dst_hbm.at[pl.ds(0, IB)], didx_v.at[slot], lsem[slot]
            ).wait()
            if tb + 1 < NB_I:
                nb_base = rowbase + (tb + 1) * IB
                pltpu.async_copy(
                    src_hbm.at[pl.ds(nb_base, IB)], sidx_v.at[1 - slot], lsem[1 - slot])
                pltpu.async_copy(
                    dst_hbm.at[pl.ds(nb_base, IB)], didx_v.at[1 - slot], lsem[1 - slot])

            @pl.loop(0, IB // NBUF)
            def _(u):
                gd = [
                    pltpu.async_copy(
                        h_hbm.at[sidx_v.at[slot, u * NBUF + b]], rows_v.at[b], gsem[b])
                    for b in range(NBUF)
                ]
                sd = []
                for b in range(NBUF):
                    gd[b].wait()
                    sd.append(pltpu.async_copy(
                        rows_v.at[b], acc_sh.at[didx_v.at[slot, u * NBUF + b]],
                        ssem[b], add=True))
                for d in sd:
                    d.wait()

        plsc.subcore_barrier()
        pltpu.sync_copy(
            acc_sh.at[pl.ds(rb, rpt)],
            out_hbm.at[pl.ds(cid * N_pad + rb, rpt)],
        )

    return agg_kernel


def _h_body(deg_ref, f_ref, h_ref):
    dsrc = deg_ref[0, 0, :] + deg_ref[1, 0, :]
    norm = lax.rsqrt(jnp.maximum(dsrc, 1.0))
    h_ref[...] = f_ref[...] * norm[:, None]


def _final_body(aggp_ref, deg_ref, f_ref, wc_ref, bc_ref, wa_ref, ba_ref, o_ref):
    agg = aggp_ref[0] + aggp_ref[1]
    din = deg_ref[0, 1, :] + deg_ref[1, 1, :]
    norm = lax.rsqrt(jnp.maximum(din, 1.0))
    agg = agg * norm[:, None]
    conv = jnp.dot(agg, wc_ref[...], preferred_element_type=jnp.float32)
    conv = conv + bc_ref[...][None, :]
    D = f_ref.shape[1]
    out = jnp.dot(conv, wa_ref[0:D, :], preferred_element_type=jnp.float32)
    out = out + jnp.dot(f_ref[...], wa_ref[D : 2 * D, :], preferred_element_type=jnp.float32)
    o_ref[...] = out + ba_ref[...][None, :]


def kernel(features, edge_index, W_conv, b_conv, W_aggr, b_aggr):
    N, D = features.shape
    E = edge_index.shape[1]
    N_pad = ((N + 2 * NW * 16 - 1) // (2 * NW * 16)) * (2 * NW * 16)
    E_pad = NW * KR * CH
    BLK = 1024

    src = edge_index[0]
    dst = edge_index[1]
    trash = N + (jnp.arange(E_pad - E, dtype=jnp.int32) % (N_pad - N))
    src_p = jnp.concatenate([src, trash]).reshape(NW * KR, CH)
    dst_p = jnp.concatenate([dst, trash]).reshape(NW * KR, CH)

    deg = _build_deg(N_pad)(src_p, dst_p).reshape(NC, 2, N_pad)
    fpad = jnp.pad(features, ((0, N_pad - N), (0, 0)))

    h = pl.pallas_call(
        _h_body,
        grid=(N_pad // BLK,),
        in_specs=[
            pl.BlockSpec((NC, 2, BLK), lambda i: (0, 0, i)),
            pl.BlockSpec((BLK, D), lambda i: (i, 0)),
        ],
        out_specs=pl.BlockSpec((BLK, D), lambda i: (i, 0)),
        out_shape=jax.ShapeDtypeStruct((N_pad, D), jnp.float32),
    )(deg, fpad)

    aggp = _build_agg(N_pad, D)(h, src_p, dst_p).reshape(NC, N_pad, D)

    out = pl.pallas_call(
        _final_body,
        grid=(N_pad // BLK,),
        in_specs=[
            pl.BlockSpec((NC, BLK, D), lambda i: (0, i, 0)),
            pl.BlockSpec((NC, 2, BLK), lambda i: (0, 0, i)),
            pl.BlockSpec((BLK, D), lambda i: (i, 0)),
            pl.BlockSpec((D, D), lambda i: (0, 0)),
            pl.BlockSpec((D,), lambda i: (0,)),
            pl.BlockSpec((2 * D, D), lambda i: (0, 0)),
            pl.BlockSpec((D,), lambda i: (0,)),
        ],
        out_specs=pl.BlockSpec((BLK, D), lambda i: (i, 0)),
        out_shape=jax.ShapeDtypeStruct((N_pad, D), jnp.float32),
    )(aggp, deg, fpad, W_conv, b_conv, W_aggr, b_aggr)

    return out[:N]


# speedup vs baseline: 10.8086x; 1.0138x over previous
import functools

import jax
import jax.numpy as jnp
from jax import lax
from jax.experimental import pallas as pl
from jax.experimental.pallas import tpu as pltpu
from jax.experimental.pallas import tpu_sc as plsc

NC = 2
NS = 16
NW = NC * NS
CH = 128
KR = 80
NBUF = 2


def _mesh():
    return plsc.VectorSubcoreMesh(
        core_axis_name="c", subcore_axis_name="s", num_cores=NC, num_subcores=NS
    )


def _build_deg(N_pad):
    spt = N_pad // NS
    BK = 8

    @functools.partial(
        pl.kernel,
        mesh=_mesh(),
        out_type=jax.ShapeDtypeStruct((NC * 2 * N_pad,), jnp.float32),
        scratch_types=[
            pltpu.VMEM((KR, CH), jnp.int32),
            pltpu.VMEM((KR, CH), jnp.int32),
            pltpu.VMEM((CH,), jnp.float32),
            pltpu.VMEM((spt,), jnp.float32),
            pltpu.VMEM_SHARED((N_pad,), jnp.float32),
            pltpu.VMEM_SHARED((N_pad,), jnp.float32),
            pltpu.SemaphoreType.DMA,
            pltpu.SemaphoreType.DMA,
        ],
    )
    def deg_kernel(src_hbm, dst_hbm, out_hbm, sidx_v, didx_v, ones_v, zslice_v,
                   dsrc_sh, ddst_sh, lsem, ssem):
        cid = lax.axis_index("c")
        sid = lax.axis_index("s")
        w = cid * NS + sid

        ld_s = pltpu.async_copy(src_hbm.at[pl.ds(w * KR, KR)], sidx_v, lsem)
        ld_d = pltpu.async_copy(dst_hbm.at[pl.ds(w * KR, KR)], didx_v, lsem)

        @pl.loop(0, CH // 16)
        def _(i):
            ones_v[pl.ds(i * 16, 16)] = jnp.full((16,), 1.0, jnp.float32)

        @pl.loop(0, spt // 16)
        def _(i):
            zslice_v[pl.ds(i * 16, 16)] = jnp.zeros((16,), jnp.float32)

        nb = sid * spt
        pltpu.sync_copy(zslice_v, dsrc_sh.at[pl.ds(nb, spt)])
        pltpu.sync_copy(zslice_v, ddst_sh.at[pl.ds(nb, spt)])
        ld_s.wait()
        ld_d.wait()
        plsc.subcore_barrier()

        @pl.loop(0, KR // BK)
        def _(t):
            descs = []
            for b in range(BK):
                descs.append(pltpu.async_copy(
                    ones_v, dsrc_sh.at[sidx_v.at[t * BK + b]], ssem, add=True))
                descs.append(pltpu.async_copy(
                    ones_v, ddst_sh.at[didx_v.at[t * BK + b]], ssem, add=True))
            for d in descs:
                d.wait()

        plsc.subcore_barrier()
        pltpu.sync_copy(
            dsrc_sh.at[pl.ds(nb, spt)],
            out_hbm.at[pl.ds((cid * 2 + 0) * N_pad + nb, spt)],
        )
        pltpu.sync_copy(
            ddst_sh.at[pl.ds(nb, spt)],
            out_hbm.at[pl.ds((cid * 2 + 1) * N_pad + nb, spt)],
        )

    return deg_kernel


def _build_agg(N_pad, D):
    rpt = N_pad // NS
    IB = 16
    NB_I = KR // IB

    @functools.partial(
        pl.kernel,
        mesh=_mesh(),
        out_type=jax.ShapeDtypeStruct((NC * N_pad, D), jnp.float32),
        scratch_types=[
            pltpu.VMEM((2, IB, CH), jnp.int32),
            pltpu.VMEM((2, IB, CH), jnp.int32),
            pltpu.VMEM((NBUF, CH, D), jnp.float32),
            pltpu.VMEM_SHARED((N_pad, D), jnp.float32),
            pltpu.SemaphoreType.DMA,
            pltpu.SemaphoreType.DMA,
        ]
        + [pltpu.SemaphoreType.DMA] * NBUF
        + [pltpu.SemaphoreType.DMA] * NBUF,
    )
    def agg_kernel(h_hbm, src_hbm, dst_hbm, out_hbm, sidx_v, didx_v, rows_v,
                   acc_sh, *sems):
        lsem = sems[:2]
        gsem = sems[2 : 2 + NBUF]
        ssem = sems[2 + NBUF :]
        cid = lax.axis_index("c")
        sid = lax.axis_index("s")
        w = cid * NS + sid
        rowbase = w * KR

        pltpu.async_copy(src_hbm.at[pl.ds(rowbase, IB)], sidx_v.at[0], lsem[0])
        pltpu.async_copy(dst_hbm.at[pl.ds(rowbase, IB)], didx_v.at[0], lsem[0])

        @pl.loop(0, CH)
        def _(r):
            for k in range(D // 16):
                rows_v[0, r, pl.ds(k * 16, 16)] = jnp.zeros((16,), jnp.float32)

        rb = sid * rpt

        @pl.loop(0, rpt // CH)
        def _(t):
            pltpu.sync_copy(rows_v.at[0], acc_sh.at[pl.ds(rb + t * CH, CH)])

        plsc.subcore_barrier()

        for tb in range(NB_I):
            slot = tb % 2
            pltpu.make_async_copy(
                src_hbm.at[pl.ds(0, IB)], sidx_v.at[slot], lsem[slot]
            ).wait()
            pltpu.make_async_copy(
                dst_hbm.at[pl.ds(0, IB)], didx_v.at[slot], lsem[slot]
            ).wait()
            if tb + 1 < NB_I:
                nb_base = rowbase + (tb + 1) * IB
                pltpu.async_copy(
                    src_hbm.at[pl.ds(nb_base, IB)], sidx_v.at[1 - slot], lsem[1 - slot])
                pltpu.async_copy(
                    dst_hbm.at[pl.ds(nb_base, IB)], didx_v.at[1 - slot], lsem[1 - slot])

            if tb == 0:
                for b in range(NBUF):
                    pltpu.async_copy(
                        h_hbm.at[sidx_v.at[slot, b]], rows_v.at[b], gsem[b])
                for b in range(NBUF):
                    pltpu.make_async_copy(
                        h_hbm.at[pl.ds(0, CH)], rows_v.at[b], gsem[b]).wait()
                    pltpu.async_copy(
                        rows_v.at[b], acc_sh.at[didx_v.at[slot, b]], ssem[b], add=True)
                lo = 1
            else:
                lo = 0

            @pl.loop(lo, IB // NBUF)
            def _(u):
                for b in range(NBUF):
                    pltpu.make_async_copy(
                        rows_v.at[b], acc_sh.at[pl.ds(0, CH)], ssem[b]).wait()
                    pltpu.async_copy(
                        h_hbm.at[sidx_v.at[slot, u * NBUF + b]], rows_v.at[b], gsem[b])
                for b in range(NBUF):
                    pltpu.make_async_copy(
                        h_hbm.at[pl.ds(0, CH)], rows_v.at[b], gsem[b]).wait()
                    pltpu.async_copy(
                        rows_v.at[b], acc_sh.at[didx_v.at[slot, u * NBUF + b]],
                        ssem[b], add=True)

        for b in range(NBUF):
            pltpu.make_async_copy(rows_v.at[b], acc_sh.at[pl.ds(0, CH)], ssem[b]).wait()

        plsc.subcore_barrier()
        pltpu.sync_copy(
            acc_sh.at[pl.ds(rb, rpt)],
            out_hbm.at[pl.ds(cid * N_pad + rb, rpt)],
        )

    return agg_kernel


def _h_body(deg_ref, f_ref, h_ref):
    dsrc = deg_ref[0, 0, :] + deg_ref[1, 0, :]
    norm = lax.rsqrt(jnp.maximum(dsrc, 1.0))
    h_ref[...] = f_ref[...] * norm[:, None]


def _final_body(aggp_ref, deg_ref, f_ref, wc_ref, bc_ref, wa_ref, ba_ref, o_ref):
    agg = aggp_ref[0] + aggp_ref[1]
    din = deg_ref[0, 1, :] + deg_ref[1, 1, :]
    norm = lax.rsqrt(jnp.maximum(din, 1.0))
    agg = agg * norm[:, None]
    conv = jnp.dot(agg, wc_ref[...], preferred_element_type=jnp.float32)
    conv = conv + bc_ref[...][None, :]
    D = f_ref.shape[1]
    out = jnp.dot(conv, wa_ref[0:D, :], preferred_element_type=jnp.float32)
    out = out + jnp.dot(f_ref[...], wa_ref[D : 2 * D, :], preferred_element_type=jnp.float32)
    o_ref[...] = out + ba_ref[...][None, :]


def kernel(features, edge_index, W_conv, b_conv, W_aggr, b_aggr):
    N, D = features.shape
    E = edge_index.shape[1]
    N_pad = ((N + 2 * NW * 16 - 1) // (2 * NW * 16)) * (2 * NW * 16)
    E_pad = NW * KR * CH
    BLK = 1024

    src = edge_index[0]
    dst = edge_index[1]
    trash = N + (jnp.arange(E_pad - E, dtype=jnp.int32) % (N_pad - N))
    src_p = jnp.concatenate([src, trash]).reshape(NW * KR, CH)
    dst_p = jnp.concatenate([dst, trash]).reshape(NW * KR, CH)

    deg = _build_deg(N_pad)(src_p, dst_p).reshape(NC, 2, N_pad)
    fpad = jnp.pad(features, ((0, N_pad - N), (0, 0)))

    h = pl.pallas_call(
        _h_body,
        grid=(N_pad // BLK,),
        in_specs=[
            pl.BlockSpec((NC, 2, BLK), lambda i: (0, 0, i)),
            pl.BlockSpec((BLK, D), lambda i: (i, 0)),
        ],
        out_specs=pl.BlockSpec((BLK, D), lambda i: (i, 0)),
        out_shape=jax.ShapeDtypeStruct((N_pad, D), jnp.float32),
    )(deg, fpad)

    aggp = _build_agg(N_pad, D)(h, src_p, dst_p).reshape(NC, N_pad, D)

    out = pl.pallas_call(
        _final_body,
        grid=(N_pad // BLK,),
        in_specs=[
            pl.BlockSpec((NC, BLK, D), lambda i: (0, i, 0)),
            pl.BlockSpec((NC, 2, BLK), lambda i: (0, 0, i)),
            pl.BlockSpec((BLK, D), lambda i: (i, 0)),
            pl.BlockSpec((D, D), lambda i: (0, 0)),
            pl.BlockSpec((D,), lambda i: (0,)),
            pl.BlockSpec((2 * D, D), lambda i: (0, 0)),
            pl.BlockSpec((D,), lambda i: (0,)),
        ],
        out_specs=pl.BlockSpec((BLK, D), lambda i: (i, 0)),
        out_shape=jax.ShapeDtypeStruct((N_pad, D), jnp.float32),
    )(aggp, deg, fpad, W_conv, b_conv, W_aggr, b_aggr)

    return out[:N]


# speedup vs baseline: 10.8165x; 1.0007x over previous
import functools

import jax
import jax.numpy as jnp
from jax import lax
from jax.experimental import pallas as pl
from jax.experimental.pallas import tpu as pltpu
from jax.experimental.pallas import tpu_sc as plsc

NC = 2
NS = 16
NW = NC * NS
CH = 128
KR = 80
NBUF = 2


def _mesh():
    return plsc.VectorSubcoreMesh(
        core_axis_name="c", subcore_axis_name="s", num_cores=NC, num_subcores=NS
    )


def _build_deg(N_pad):
    spt = N_pad // NS
    BK = 8

    @functools.partial(
        pl.kernel,
        mesh=_mesh(),
        out_type=jax.ShapeDtypeStruct((NC * 2 * N_pad,), jnp.float32),
        scratch_types=[
            pltpu.VMEM((KR, CH), jnp.int32),
            pltpu.VMEM((KR, CH), jnp.int32),
            pltpu.VMEM((CH,), jnp.float32),
            pltpu.VMEM((spt,), jnp.float32),
            pltpu.VMEM_SHARED((N_pad,), jnp.float32),
            pltpu.VMEM_SHARED((N_pad,), jnp.float32),
            pltpu.SemaphoreType.DMA,
            pltpu.SemaphoreType.DMA,
        ],
    )
    def deg_kernel(src_hbm, dst_hbm, out_hbm, sidx_v, didx_v, ones_v, zslice_v,
                   dsrc_sh, ddst_sh, lsem, ssem):
        cid = lax.axis_index("c")
        sid = lax.axis_index("s")
        w = cid * NS + sid

        ld_s = pltpu.async_copy(src_hbm.at[pl.ds(w * KR, KR)], sidx_v, lsem)
        ld_d = pltpu.async_copy(dst_hbm.at[pl.ds(w * KR, KR)], didx_v, lsem)

        @pl.loop(0, CH // 16)
        def _(i):
            ones_v[pl.ds(i * 16, 16)] = jnp.full((16,), 1.0, jnp.float32)

        @pl.loop(0, spt // 16)
        def _(i):
            zslice_v[pl.ds(i * 16, 16)] = jnp.zeros((16,), jnp.float32)

        nb = sid * spt
        pltpu.sync_copy(zslice_v, dsrc_sh.at[pl.ds(nb, spt)])
        pltpu.sync_copy(zslice_v, ddst_sh.at[pl.ds(nb, spt)])
        ld_s.wait()
        ld_d.wait()
        plsc.subcore_barrier()

        @pl.loop(0, KR // BK)
        def _(t):
            descs = []
            for b in range(BK):
                descs.append(pltpu.async_copy(
                    ones_v, dsrc_sh.at[sidx_v.at[t * BK + b]], ssem, add=True))
                descs.append(pltpu.async_copy(
                    ones_v, ddst_sh.at[didx_v.at[t * BK + b]], ssem, add=True))
            for d in descs:
                d.wait()

        plsc.subcore_barrier()
        pltpu.sync_copy(
            dsrc_sh.at[pl.ds(nb, spt)],
            out_hbm.at[pl.ds((cid * 2 + 0) * N_pad + nb, spt)],
        )
        pltpu.sync_copy(
            ddst_sh.at[pl.ds(nb, spt)],
            out_hbm.at[pl.ds((cid * 2 + 1) * N_pad + nb, spt)],
        )

    return deg_kernel


def _build_agg(N_pad, D):
    rpt = N_pad // NS
    IB = 16
    NB_I = KR // IB

    @functools.partial(
        pl.kernel,
        mesh=_mesh(),
        out_type=jax.ShapeDtypeStruct((NC * N_pad, D), jnp.float32),
        scratch_types=[
            pltpu.VMEM((2, IB, CH), jnp.int32),
            pltpu.VMEM((2, IB, CH), jnp.int32),
            pltpu.VMEM((NBUF, CH, D), jnp.float32),
            pltpu.VMEM_SHARED((N_pad, D), jnp.float32),
            pltpu.SemaphoreType.DMA,
            pltpu.SemaphoreType.DMA,
        ]
        + [pltpu.SemaphoreType.DMA] * NBUF
        + [pltpu.SemaphoreType.DMA] * NBUF,
    )
    def agg_kernel(h_hbm, src_hbm, dst_hbm, out_hbm, sidx_v, didx_v, rows_v,
                   acc_sh, *sems):
        lsem = sems[:2]
        gsem = sems[2 : 2 + NBUF]
        ssem = sems[2 + NBUF :]
        cid = lax.axis_index("c")
        sid = lax.axis_index("s")
        w = cid * NS + sid
        rowbase = w * KR

        pltpu.async_copy(src_hbm.at[pl.ds(rowbase, IB)], sidx_v.at[0], lsem[0])
        pltpu.async_copy(dst_hbm.at[pl.ds(rowbase, IB)], didx_v.at[0], lsem[0])

        @pl.loop(0, CH)
        def _(r):
            for k in range(D // 16):
                rows_v[0, r, pl.ds(k * 16, 16)] = jnp.zeros((16,), jnp.float32)

        rb = sid * rpt

        @pl.loop(0, rpt // CH)
        def _(t):
            pltpu.sync_copy(rows_v.at[0], acc_sh.at[pl.ds(rb + t * CH, CH)])

        plsc.subcore_barrier()

        for tb in range(NB_I):
            slot = tb % 2
            pltpu.make_async_copy(
                src_hbm.at[pl.ds(0, IB)], sidx_v.at[slot], lsem[slot]
            ).wait()
            pltpu.make_async_copy(
                dst_hbm.at[pl.ds(0, IB)], didx_v.at[slot], lsem[slot]
            ).wait()
            if tb + 1 < NB_I:
                nb_base = rowbase + (tb + 1) * IB
                pltpu.async_copy(
                    src_hbm.at[pl.ds(nb_base, IB)], sidx_v.at[1 - slot], lsem[1 - slot])
                pltpu.async_copy(
                    dst_hbm.at[pl.ds(nb_base, IB)], didx_v.at[1 - slot], lsem[1 - slot])

            def fire_gather(b, r):
                pltpu.async_copy(
                    h_hbm.at[sidx_v.at[slot, r, pl.ds(0, CH // 2)]],
                    rows_v.at[b, pl.ds(0, CH // 2)], gsem[b])
                pltpu.async_copy(
                    h_hbm.at[sidx_v.at[slot, r, pl.ds(CH // 2, CH // 2)]],
                    rows_v.at[b, pl.ds(CH // 2, CH // 2)], gsem[b])

            def wait_gather(b):
                pltpu.make_async_copy(
                    h_hbm.at[pl.ds(0, CH)], rows_v.at[b], gsem[b]).wait()

            if tb == 0:
                for b in range(NBUF):
                    fire_gather(b, b)
                for b in range(NBUF):
                    wait_gather(b)
                    pltpu.async_copy(
                        rows_v.at[b], acc_sh.at[didx_v.at[slot, b]], ssem[b], add=True)
                lo = 1
            else:
                lo = 0

            @pl.loop(lo, IB // NBUF)
            def _(u):
                for b in range(NBUF):
                    pltpu.make_async_copy(
                        rows_v.at[b], acc_sh.at[pl.ds(0, CH)], ssem[b]).wait()
                    fire_gather(b, u * NBUF + b)
                for b in range(NBUF):
                    wait_gather(b)
                    pltpu.async_copy(
                        rows_v.at[b], acc_sh.at[didx_v.at[slot, u * NBUF + b]],
                        ssem[b], add=True)

        for b in range(NBUF):
            pltpu.make_async_copy(rows_v.at[b], acc_sh.at[pl.ds(0, CH)], ssem[b]).wait()

        plsc.subcore_barrier()
        pltpu.sync_copy(
            acc_sh.at[pl.ds(rb, rpt)],
            out_hbm.at[pl.ds(cid * N_pad + rb, rpt)],
        )

    return agg_kernel


def _h_body(deg_ref, f_ref, h_ref):
    dsrc = deg_ref[0, 0, :] + deg_ref[1, 0, :]
    norm = lax.rsqrt(jnp.maximum(dsrc, 1.0))
    h_ref[...] = f_ref[...] * norm[:, None]


def _final_body(aggp_ref, deg_ref, f_ref, wc_ref, bc_ref, wa_ref, ba_ref, o_ref):
    agg = aggp_ref[0] + aggp_ref[1]
    din = deg_ref[0, 1, :] + deg_ref[1, 1, :]
    norm = lax.rsqrt(jnp.maximum(din, 1.0))
    agg = agg * norm[:, None]
    conv = jnp.dot(agg, wc_ref[...], preferred_element_type=jnp.float32)
    conv = conv + bc_ref[...][None, :]
    D = f_ref.shape[1]
    out = jnp.dot(conv, wa_ref[0:D, :], preferred_element_type=jnp.float32)
    out = out + jnp.dot(f_ref[...], wa_ref[D : 2 * D, :], preferred_element_type=jnp.float32)
    o_ref[...] = out + ba_ref[...][None, :]


def kernel(features, edge_index, W_conv, b_conv, W_aggr, b_aggr):
    N, D = features.shape
    E = edge_index.shape[1]
    N_pad = ((N + 2 * NW * 16 - 1) // (2 * NW * 16)) * (2 * NW * 16)
    E_pad = NW * KR * CH
    BLK = 1024

    src = edge_index[0]
    dst = edge_index[1]
    trash = N + (jnp.arange(E_pad - E, dtype=jnp.int32) % (N_pad - N))
    src_p = jnp.concatenate([src, trash]).reshape(NW * KR, CH)
    dst_p = jnp.concatenate([dst, trash]).reshape(NW * KR, CH)

    deg = _build_deg(N_pad)(src_p, dst_p).reshape(NC, 2, N_pad)
    fpad = jnp.pad(features, ((0, N_pad - N), (0, 0)))

    h = pl.pallas_call(
        _h_body,
        grid=(N_pad // BLK,),
        in_specs=[
            pl.BlockSpec((NC, 2, BLK), lambda i: (0, 0, i)),
            pl.BlockSpec((BLK, D), lambda i: (i, 0)),
        ],
        out_specs=pl.BlockSpec((BLK, D), lambda i: (i, 0)),
        out_shape=jax.ShapeDtypeStruct((N_pad, D), jnp.float32),
    )(deg, fpad)

    aggp = _build_agg(N_pad, D)(h, src_p, dst_p).reshape(NC, N_pad, D)

    out = pl.pallas_call(
        _final_body,
        grid=(N_pad // BLK,),
        in_specs=[
            pl.BlockSpec((NC, BLK, D), lambda i: (0, i, 0)),
            pl.BlockSpec((NC, 2, BLK), lambda i: (0, 0, i)),
            pl.BlockSpec((BLK, D), lambda i: (i, 0)),
            pl.BlockSpec((D, D), lambda i: (0, 0)),
            pl.BlockSpec((D,), lambda i: (0,)),
            pl.BlockSpec((2 * D, D), lambda i: (0, 0)),
            pl.BlockSpec((D,), lambda i: (0,)),
        ],
        out_specs=pl.BlockSpec((BLK, D), lambda i: (i, 0)),
        out_shape=jax.ShapeDtypeStruct((N_pad, D), jnp.float32),
    )(aggp, deg, fpad, W_conv, b_conv, W_aggr, b_aggr)

    return out[:N]
